# Initial kernel scaffold; baseline (speedup 1.0000x reference)
import jax
import jax.numpy as jnp
from jax.experimental import pallas as pl


def kernel(sample, ber_weight, mix_weight):
    raise NotImplementedError("write your pallas kernel here")


# speedup vs baseline: 5.9604x; 5.9604x over previous
import jax
import jax.numpy as jnp
from jax.experimental import pallas as pl

_EPS = 1e-07


def _body(s_ref, bw_ref, mw_ref, o_ref):
    bw = bw_ref[...]
    p = jax.nn.sigmoid(bw)
    a = jnp.log(p + _EPS)
    c = jnp.log(1.0 - p + _EPS)
    d = a - c
    u = 0.5 * jnp.sum(a + c, axis=1)
    mw = mw_ref[0, :]
    mixp = jnp.exp(mw - jnp.max(mw))
    mixp = mixp / jnp.sum(mixp)
    umax = jnp.max(u)
    coef = mixp * jnp.exp(u - umax)
    t = 0.5 * jax.lax.dot_general(
        s_ref[...], d, (((1,), (1,)), ((), ())),
        preferred_element_type=jnp.float32)
    e = jnp.exp(t) + jnp.exp(-t)
    acc = jnp.sum(coef[None, :] * e, axis=1)
    o_ref[...] = jnp.log(0.5 * acc) + umax


def kernel(sample, ber_weight, mix_weight):
    b = sample.shape[0]
    s2 = sample.reshape(b, -1)
    w, s = ber_weight.shape[0], s2.shape[1]
    bw2 = ber_weight.reshape(w, s)
    mw2 = mix_weight.reshape(1, w)
    return pl.pallas_call(
        _body,
        out_shape=jax.ShapeDtypeStruct((b,), jnp.float32),
    )(s2, bw2, mw2)
